# Initial kernel scaffold; baseline (speedup 1.0000x reference)
import functools

import jax
import jax.numpy as jnp
from jax import lax
from jax.experimental import pallas as pl
from jax.experimental.pallas import tpu as pltpu
from jax.experimental.pallas import tpu_sc as plsc

N = 10000
E = 320000
D = 128
R = 64
B = 2

NPAD = 10240
NTILE = 16
EPT = E // NTILE
VECS = EPT // 16
NODES_PER_TILE = NPAD // NTILE


def _prologue_body(ri_ref, rel_ref, w_ref, aux_ref):
    ri = ri_ref[...]
    rel = rel_ref[...]
    w = w_ref[...]
    iota_r = lax.broadcasted_iota(jnp.int32, (8, R), 1)
    oh = (ri[:, :R] == iota_r).astype(jnp.float32)
    query = jnp.dot(oh, rel, preferred_element_type=jnp.float32)
    wq = lax.dot_general(query, w, (((1,), (1,)), ((), ())),
                         preferred_element_type=jnp.float32)
    coef = lax.dot_general(query * wq, rel, (((1,), (1,)), ((), ())),
                           preferred_element_type=jnp.float32)
    q2 = jnp.sum(query * query, axis=1, keepdims=True)
    aux_ref[...] = jnp.concatenate(
        [coef, q2, jnp.zeros((8, D - R - 1), jnp.float32)], axis=1)


def _epilogue_body(ssum_ref, smax_ref, h_ref, aux_ref, out_ref):
    q2 = aux_ref[0:2, 64:65]
    score = ssum_ref[...] + h_ref[...] * q2
    neural = 1.0 / (1.0 + jnp.exp(-score))
    sym = jnp.maximum(smax_ref[...], 0.0)
    t = neural * sym
    out_ref[...] = jnp.log((t + 1e-10) / (1.0 - t + 1e-10))


def _sc_body(h_hbm, src_hbm, dst_hbm, et_hbm, aux_hbm, rb_hbm,
             osum_hbm, omax_hbm,
             hv, cv, rvv, srcv, dstv, etv, sumacc, maxacc, done_s,
             bsum, bmax, osum_v, omax_v, shsum, shmax):
    b = lax.axis_index("c")
    s = lax.axis_index("s")

    pltpu.sync_copy(h_hbm.at[b], hv)
    pltpu.sync_copy(aux_hbm.at[b, pl.ds(0, R)], cv)
    pltpu.sync_copy(rb_hbm.at[b], rvv)
    ebase = s * EPT
    pltpu.sync_copy(src_hbm.at[pl.ds(ebase, EPT)], srcv)
    pltpu.sync_copy(dst_hbm.at[pl.ds(ebase, EPT)], dstv)
    pltpu.sync_copy(et_hbm.at[pl.ds(ebase, EPT)], etv)

    zeros16 = jnp.zeros((16,), jnp.float32)
    neg16 = jnp.full((16,), -1e9, jnp.float32)

    def init_body(i, c):
        sumacc[pl.ds(i * 16, 16)] = zeros16
        maxacc[pl.ds(i * 16, 16)] = neg16
        return c

    lax.fori_loop(0, NPAD // 16, init_body, 0)

    rv = rvv[...]

    def edge_body(i, c):
        off = i * 16
        s16 = srcv[pl.ds(off, 16)]
        d16 = dstv[pl.ds(off, 16)]
        t16 = etv[pl.ds(off, 16)]
        hb = plsc.load_gather(hv, [s16])
        cf = plsc.load_gather(cv, [t16])
        plsc.addupdate_scatter(sumacc, [d16], hb * cf)

        m = t16 == rv

        @pl.when(jnp.any(m))
        def _():
            cur = plsc.load_gather(maxacc, [d16])
            plsc.store_scatter(maxacc, [d16], jnp.maximum(cur, hb), mask=m)
            chk = plsc.load_gather(maxacc, [d16])
            pend = m & (chk < hb)
            done_s[...] = jnp.where(pend, 0, 1)

            @pl.when(jnp.any(pend))
            def _():
                def retry(k, c2):
                    p = done_s[...] == 0

                    @pl.when(jnp.any(p))
                    def _():
                        cur2 = plsc.load_gather(maxacc, [d16])
                        plsc.store_scatter(maxacc, [d16],
                                           jnp.maximum(cur2, hb), mask=p)
                        chk2 = plsc.load_gather(maxacc, [d16])
                        done_s[...] = jnp.where(p & (chk2 < hb), 0, 1)

                    return c2

                lax.fori_loop(0, 15, retry, 0)

        return c

    lax.fori_loop(0, VECS, edge_body, 0)

    pltpu.sync_copy(sumacc, shsum.at[s])
    pltpu.sync_copy(maxacc, shmax.at[s])
    plsc.subcore_barrier()

    nbase = s * NODES_PER_TILE
    for j in range(NTILE):
        pltpu.sync_copy(shsum.at[j, pl.ds(nbase, NODES_PER_TILE)], bsum.at[j])
        pltpu.sync_copy(shmax.at[j, pl.ds(nbase, NODES_PER_TILE)], bmax.at[j])

    def comb_body(k, c):
        off = k * 16
        a = bsum[0, pl.ds(off, 16)]
        mx = bmax[0, pl.ds(off, 16)]
        for j in range(1, NTILE):
            a = a + bsum[j, pl.ds(off, 16)]
            mx = jnp.maximum(mx, bmax[j, pl.ds(off, 16)])
        osum_v[pl.ds(off, 16)] = a
        omax_v[pl.ds(off, 16)] = mx
        return c

    lax.fori_loop(0, NODES_PER_TILE // 16, comb_body, 0)
    pltpu.sync_copy(osum_v, osum_hbm.at[b, pl.ds(nbase, NODES_PER_TILE)])
    pltpu.sync_copy(omax_v, omax_hbm.at[b, pl.ds(nbase, NODES_PER_TILE)])


def _build_sc_call():
    mesh = plsc.VectorSubcoreMesh(core_axis_name="c", subcore_axis_name="s")
    return pl.kernel(
        _sc_body,
        out_type=(jax.ShapeDtypeStruct((B, NPAD), jnp.float32),
                  jax.ShapeDtypeStruct((B, NPAD), jnp.float32)),
        mesh=mesh,
        scratch_types=[
            pltpu.VMEM((N,), jnp.float32),
            pltpu.VMEM((R,), jnp.float32),
            pltpu.VMEM((16,), jnp.int32),
            pltpu.VMEM((EPT,), jnp.int32),
            pltpu.VMEM((EPT,), jnp.int32),
            pltpu.VMEM((EPT,), jnp.int32),
            pltpu.VMEM((NPAD,), jnp.float32),
            pltpu.VMEM((NPAD,), jnp.float32),
            pltpu.VMEM((16,), jnp.int32),
            pltpu.VMEM((NTILE, NODES_PER_TILE), jnp.float32),
            pltpu.VMEM((NTILE, NODES_PER_TILE), jnp.float32),
            pltpu.VMEM((NODES_PER_TILE,), jnp.float32),
            pltpu.VMEM((NODES_PER_TILE,), jnp.float32),
            pltpu.VMEM_SHARED((NTILE, NPAD), jnp.float32),
            pltpu.VMEM_SHARED((NTILE, NPAD), jnp.float32),
        ],
    )


def kernel(h_prob, edge_index, edge_type, r_index, rel_emb, W):
    src = edge_index[0]
    dst = edge_index[1]
    ri32 = r_index.astype(jnp.int32)
    rb16 = jnp.broadcast_to(ri32[:, None], (B, 16))
    ri8 = jnp.concatenate(
        [jnp.broadcast_to(ri32[:, None], (B, D)),
         jnp.zeros((8 - B, D), jnp.int32)], axis=0)

    aux = pl.pallas_call(
        _prologue_body,
        out_shape=jax.ShapeDtypeStruct((8, D), jnp.float32),
    )(ri8, rel_emb, W)

    ssum_p, smax_p = _build_sc_call()(h_prob, src, dst, edge_type, aux, rb16)

    h_pad = jnp.concatenate(
        [h_prob, jnp.zeros((B, NPAD - N), jnp.float32)], axis=1)
    out_pad = pl.pallas_call(
        _epilogue_body,
        out_shape=jax.ShapeDtypeStruct((B, NPAD), jnp.float32),
    )(ssum_p, smax_p, h_pad, aux)
    return out_pad[:, :N]


# speedup vs baseline: 284.4561x; 284.4561x over previous
import functools

import jax
import jax.numpy as jnp
from jax import lax
from jax.experimental import pallas as pl
from jax.experimental.pallas import tpu as pltpu
from jax.experimental.pallas import tpu_sc as plsc

N = 10000
E = 320000
D = 128
R = 64
B = 2

NPAD = 10240
NTILE = 16
EPT = E // NTILE
ECHUNK = 4000
NCHUNK = EPT // ECHUNK
CVECS = ECHUNK // 16
NODES_PER_TILE = NPAD // NTILE


def _prologue_body(ri_ref, rel_ref, w_ref, aux_ref):
    ri = ri_ref[...]
    rel = rel_ref[...]
    w = w_ref[...]
    iota_r = lax.broadcasted_iota(jnp.int32, (8, R), 1)
    oh = (ri[:, :R] == iota_r).astype(jnp.float32)
    query = jnp.dot(oh, rel, preferred_element_type=jnp.float32)
    wq = lax.dot_general(query, w, (((1,), (1,)), ((), ())),
                         preferred_element_type=jnp.float32)
    coef = lax.dot_general(query * wq, rel, (((1,), (1,)), ((), ())),
                           preferred_element_type=jnp.float32)
    q2 = jnp.sum(query * query, axis=1, keepdims=True)
    aux_ref[...] = jnp.concatenate(
        [coef, q2, jnp.zeros((8, D - R - 1), jnp.float32)], axis=1)


def _epilogue_body(ssum_ref, smax_ref, h_ref, aux_ref, out_ref):
    q2 = aux_ref[0:2, 64:65]
    score = ssum_ref[...] + h_ref[...] * q2
    neural = 1.0 / (1.0 + jnp.exp(-score))
    sym = jnp.maximum(smax_ref[...], 0.0)
    t = neural * sym
    out_ref[...] = jnp.log((t + 1e-10) / (1.0 - t + 1e-10))


def _sc_body(h_hbm, src_hbm, dst_hbm, et_hbm, aux_hbm, rb_hbm,
             osum_hbm, omax_hbm,
             hv, cv, rvv, srcv, dstv, etv, sumacc, maxacc, done_s,
             bsum, bmax, osum_v, omax_v, shsum, shmax):
    b = lax.axis_index("c")
    s = lax.axis_index("s")

    pltpu.sync_copy(h_hbm.at[b], hv)
    pltpu.sync_copy(aux_hbm.at[b, pl.ds(0, R)], cv)
    pltpu.sync_copy(rb_hbm.at[b], rvv)
    ebase = s * EPT

    zeros16 = jnp.zeros((16,), jnp.float32)
    neg16 = jnp.full((16,), -1e9, jnp.float32)

    def init_body(i, c):
        sumacc[pl.ds(i * 16, 16)] = zeros16
        maxacc[pl.ds(i * 16, 16)] = neg16
        return c

    lax.fori_loop(0, NPAD // 16, init_body, 0)

    rv = rvv[...]

    def edge_body(i, c):
        off = i * 16
        s16 = srcv[pl.ds(off, 16)]
        d16 = dstv[pl.ds(off, 16)]
        t16 = etv[pl.ds(off, 16)]
        hb = plsc.load_gather(hv, [s16])
        cf = plsc.load_gather(cv, [t16])
        plsc.addupdate_scatter(sumacc, [d16], hb * cf)

        m = t16 == rv

        @pl.when(jnp.any(m))
        def _():
            cur = plsc.load_gather(maxacc, [d16])
            plsc.store_scatter(maxacc, [d16], jnp.maximum(cur, hb), mask=m)
            chk = plsc.load_gather(maxacc, [d16])
            pend = m & (chk < hb)
            done_s[...] = jnp.where(pend, 0, 1)

            @pl.when(jnp.any(pend))
            def _():
                def retry(k, c2):
                    p = done_s[...] == 0

                    @pl.when(jnp.any(p))
                    def _():
                        cur2 = plsc.load_gather(maxacc, [d16])
                        plsc.store_scatter(maxacc, [d16],
                                           jnp.maximum(cur2, hb), mask=p)
                        chk2 = plsc.load_gather(maxacc, [d16])
                        done_s[...] = jnp.where(p & (chk2 < hb), 0, 1)

                    return c2

                lax.fori_loop(0, 15, retry, 0)

        return c

    def chunk_body(ci, c):
        cbase = ebase + ci * ECHUNK
        pltpu.sync_copy(src_hbm.at[pl.ds(cbase, ECHUNK)], srcv)
        pltpu.sync_copy(dst_hbm.at[pl.ds(cbase, ECHUNK)], dstv)
        pltpu.sync_copy(et_hbm.at[pl.ds(cbase, ECHUNK)], etv)
        lax.fori_loop(0, CVECS, edge_body, 0)
        return c

    lax.fori_loop(0, NCHUNK, chunk_body, 0)

    pltpu.sync_copy(sumacc, shsum.at[s])
    pltpu.sync_copy(maxacc, shmax.at[s])
    plsc.subcore_barrier()

    nbase = s * NODES_PER_TILE
    for j in range(NTILE):
        pltpu.sync_copy(shsum.at[j, pl.ds(nbase, NODES_PER_TILE)], bsum.at[j])
        pltpu.sync_copy(shmax.at[j, pl.ds(nbase, NODES_PER_TILE)], bmax.at[j])

    def comb_body(k, c):
        off = k * 16
        a = bsum[0, pl.ds(off, 16)]
        mx = bmax[0, pl.ds(off, 16)]
        for j in range(1, NTILE):
            a = a + bsum[j, pl.ds(off, 16)]
            mx = jnp.maximum(mx, bmax[j, pl.ds(off, 16)])
        osum_v[pl.ds(off, 16)] = a
        omax_v[pl.ds(off, 16)] = mx
        return c

    lax.fori_loop(0, NODES_PER_TILE // 16, comb_body, 0)
    pltpu.sync_copy(osum_v, osum_hbm.at[b, pl.ds(nbase, NODES_PER_TILE)])
    pltpu.sync_copy(omax_v, omax_hbm.at[b, pl.ds(nbase, NODES_PER_TILE)])


def _build_sc_call():
    mesh = plsc.VectorSubcoreMesh(core_axis_name="c", subcore_axis_name="s")
    return pl.kernel(
        _sc_body,
        out_type=(jax.ShapeDtypeStruct((B, NPAD), jnp.float32),
                  jax.ShapeDtypeStruct((B, NPAD), jnp.float32)),
        mesh=mesh,
        compiler_params=pltpu.CompilerParams(needs_layout_passes=False),
        scratch_types=[
            pltpu.VMEM((N,), jnp.float32),
            pltpu.VMEM((R,), jnp.float32),
            pltpu.VMEM((16,), jnp.int32),
            pltpu.VMEM((ECHUNK,), jnp.int32),
            pltpu.VMEM((ECHUNK,), jnp.int32),
            pltpu.VMEM((ECHUNK,), jnp.int32),
            pltpu.VMEM((NPAD,), jnp.float32),
            pltpu.VMEM((NPAD,), jnp.float32),
            pltpu.VMEM((16,), jnp.int32),
            pltpu.VMEM((NTILE, NODES_PER_TILE), jnp.float32),
            pltpu.VMEM((NTILE, NODES_PER_TILE), jnp.float32),
            pltpu.VMEM((NODES_PER_TILE,), jnp.float32),
            pltpu.VMEM((NODES_PER_TILE,), jnp.float32),
            pltpu.VMEM_SHARED((NTILE, NPAD), jnp.float32),
            pltpu.VMEM_SHARED((NTILE, NPAD), jnp.float32),
        ],
    )


def kernel(h_prob, edge_index, edge_type, r_index, rel_emb, W):
    src = edge_index[0]
    dst = edge_index[1]
    ri32 = r_index.astype(jnp.int32)
    rb16 = jnp.broadcast_to(ri32[:, None], (B, 16))
    ri8 = jnp.concatenate(
        [jnp.broadcast_to(ri32[:, None], (B, D)),
         jnp.zeros((8 - B, D), jnp.int32)], axis=0)

    aux = pl.pallas_call(
        _prologue_body,
        out_shape=jax.ShapeDtypeStruct((8, D), jnp.float32),
    )(ri8, rel_emb, W)

    ssum_p, smax_p = _build_sc_call()(h_prob, src, dst, edge_type, aux, rb16)

    h_pad = jnp.concatenate(
        [h_prob, jnp.zeros((B, NPAD - N), jnp.float32)], axis=1)
    out_pad = pl.pallas_call(
        _epilogue_body,
        out_shape=jax.ShapeDtypeStruct((B, NPAD), jnp.float32),
    )(ssum_p, smax_p, h_pad, aux)
    return out_pad[:, :N]


# speedup vs baseline: 303.3583x; 1.0665x over previous
import functools

import jax
import jax.numpy as jnp
from jax import lax
from jax.experimental import pallas as pl
from jax.experimental.pallas import tpu as pltpu
from jax.experimental.pallas import tpu_sc as plsc

N = 10000
E = 320000
D = 128
R = 64
B = 2

NPAD = 10240
NTILE = 16
EPT = E // NTILE
ECHUNK = 4000
NCHUNK = EPT // ECHUNK
CVECS = ECHUNK // 16
UNROLL = 5
NODES_PER_TILE = NPAD // NTILE


def _prologue_body(ri_ref, rel_ref, w_ref, aux_ref):
    ri = ri_ref[...]
    rel = rel_ref[...]
    w = w_ref[...]
    iota_r = lax.broadcasted_iota(jnp.int32, (8, R), 1)
    oh = (ri[:, :R] == iota_r).astype(jnp.float32)
    query = jnp.dot(oh, rel, preferred_element_type=jnp.float32)
    wq = lax.dot_general(query, w, (((1,), (1,)), ((), ())),
                         preferred_element_type=jnp.float32)
    coef = lax.dot_general(query * wq, rel, (((1,), (1,)), ((), ())),
                           preferred_element_type=jnp.float32)
    q2 = jnp.sum(query * query, axis=1, keepdims=True)
    aux_ref[...] = jnp.concatenate(
        [coef, q2, jnp.zeros((8, D - R - 1), jnp.float32)], axis=1)


def _epilogue_body(ssum_ref, smax_ref, h_ref, aux_ref, out_ref):
    q2 = aux_ref[0:2, 64:65]
    score = ssum_ref[...] + h_ref[...] * q2
    neural = 1.0 / (1.0 + jnp.exp(-score))
    sym = jnp.maximum(smax_ref[...], 0.0)
    t = neural * sym
    out_ref[...] = jnp.log((t + 1e-10) / (1.0 - t + 1e-10))


def _sc_body(h_hbm, src_hbm, dst_hbm, et_hbm, aux_hbm, rb_hbm,
             osum_hbm, omax_hbm,
             hv, cv, rvv, srcv, dstv, etv, sumacc, maxacc, done_s,
             bsum, bmax, osum_v, omax_v, shsum, shmax):
    b = lax.axis_index("c")
    s = lax.axis_index("s")

    pltpu.sync_copy(h_hbm.at[b], hv)
    pltpu.sync_copy(aux_hbm.at[b, pl.ds(0, R)], cv)
    pltpu.sync_copy(rb_hbm.at[b], rvv)
    ebase = s * EPT

    zeros16 = jnp.zeros((16,), jnp.float32)
    neg16 = jnp.full((16,), -1e9, jnp.float32)

    def init_body(i, c):
        sumacc[pl.ds(i * 16, 16)] = zeros16
        maxacc[pl.ds(i * 16, 16)] = neg16
        return c

    lax.fori_loop(0, NPAD // 16, init_body, 0)

    rv = rvv[...]

    def edge_vec(off):
        s16 = srcv[pl.ds(off, 16)]
        d16 = dstv[pl.ds(off, 16)]
        t16 = etv[pl.ds(off, 16)]
        hb = plsc.load_gather(hv, [s16])
        cf = plsc.load_gather(cv, [t16])
        plsc.addupdate_scatter(sumacc, [d16], hb * cf)

        m = t16 == rv

        @pl.when(jnp.any(m))
        def _():
            cur = plsc.load_gather(maxacc, [d16])
            plsc.store_scatter(maxacc, [d16], jnp.maximum(cur, hb), mask=m)
            chk = plsc.load_gather(maxacc, [d16])
            pend = m & (chk < hb)
            done_s[...] = jnp.where(pend, 0, 1)

            @pl.when(jnp.any(pend))
            def _():
                def retry(k, c2):
                    p = done_s[...] == 0

                    @pl.when(jnp.any(p))
                    def _():
                        cur2 = plsc.load_gather(maxacc, [d16])
                        plsc.store_scatter(maxacc, [d16],
                                           jnp.maximum(cur2, hb), mask=p)
                        chk2 = plsc.load_gather(maxacc, [d16])
                        done_s[...] = jnp.where(p & (chk2 < hb), 0, 1)

                    return c2

                lax.fori_loop(0, 15, retry, 0)

    def edge_body(i, c):
        base = i * (16 * UNROLL)
        for u in range(UNROLL):
            edge_vec(base + u * 16)
        return c

    def chunk_body(ci, c):
        cbase = ebase + ci * ECHUNK
        pltpu.sync_copy(src_hbm.at[pl.ds(cbase, ECHUNK)], srcv)
        pltpu.sync_copy(dst_hbm.at[pl.ds(cbase, ECHUNK)], dstv)
        pltpu.sync_copy(et_hbm.at[pl.ds(cbase, ECHUNK)], etv)
        lax.fori_loop(0, CVECS // UNROLL, edge_body, 0)
        return c

    lax.fori_loop(0, NCHUNK, chunk_body, 0)

    pltpu.sync_copy(sumacc, shsum.at[s])
    pltpu.sync_copy(maxacc, shmax.at[s])
    plsc.subcore_barrier()

    nbase = s * NODES_PER_TILE
    pltpu.sync_copy(shsum.at[:, pl.ds(nbase, NODES_PER_TILE)], bsum)
    pltpu.sync_copy(shmax.at[:, pl.ds(nbase, NODES_PER_TILE)], bmax)

    def comb_body(k, c):
        off = k * 16
        a = bsum[0, pl.ds(off, 16)]
        mx = bmax[0, pl.ds(off, 16)]
        for j in range(1, NTILE):
            a = a + bsum[j, pl.ds(off, 16)]
            mx = jnp.maximum(mx, bmax[j, pl.ds(off, 16)])
        osum_v[pl.ds(off, 16)] = a
        omax_v[pl.ds(off, 16)] = mx
        return c

    lax.fori_loop(0, NODES_PER_TILE // 16, comb_body, 0)
    pltpu.sync_copy(osum_v, osum_hbm.at[b, pl.ds(nbase, NODES_PER_TILE)])
    pltpu.sync_copy(omax_v, omax_hbm.at[b, pl.ds(nbase, NODES_PER_TILE)])


def _build_sc_call():
    mesh = plsc.VectorSubcoreMesh(core_axis_name="c", subcore_axis_name="s")
    return pl.kernel(
        _sc_body,
        out_type=(jax.ShapeDtypeStruct((B, NPAD), jnp.float32),
                  jax.ShapeDtypeStruct((B, NPAD), jnp.float32)),
        mesh=mesh,
        compiler_params=pltpu.CompilerParams(needs_layout_passes=False),
        scratch_types=[
            pltpu.VMEM((N,), jnp.float32),
            pltpu.VMEM((R,), jnp.float32),
            pltpu.VMEM((16,), jnp.int32),
            pltpu.VMEM((ECHUNK,), jnp.int32),
            pltpu.VMEM((ECHUNK,), jnp.int32),
            pltpu.VMEM((ECHUNK,), jnp.int32),
            pltpu.VMEM((NPAD,), jnp.float32),
            pltpu.VMEM((NPAD,), jnp.float32),
            pltpu.VMEM((16,), jnp.int32),
            pltpu.VMEM((NTILE, NODES_PER_TILE), jnp.float32),
            pltpu.VMEM((NTILE, NODES_PER_TILE), jnp.float32),
            pltpu.VMEM((NODES_PER_TILE,), jnp.float32),
            pltpu.VMEM((NODES_PER_TILE,), jnp.float32),
            pltpu.VMEM_SHARED((NTILE, NPAD), jnp.float32),
            pltpu.VMEM_SHARED((NTILE, NPAD), jnp.float32),
        ],
    )


def kernel(h_prob, edge_index, edge_type, r_index, rel_emb, W):
    src = edge_index[0]
    dst = edge_index[1]
    ri32 = r_index.astype(jnp.int32)
    rb16 = jnp.broadcast_to(ri32[:, None], (B, 16))
    ri8 = jnp.concatenate(
        [jnp.broadcast_to(ri32[:, None], (B, D)),
         jnp.zeros((8 - B, D), jnp.int32)], axis=0)

    aux = pl.pallas_call(
        _prologue_body,
        out_shape=jax.ShapeDtypeStruct((8, D), jnp.float32),
    )(ri8, rel_emb, W)

    ssum_p, smax_p = _build_sc_call()(h_prob, src, dst, edge_type, aux, rb16)

    h_pad = jnp.concatenate(
        [h_prob, jnp.zeros((B, NPAD - N), jnp.float32)], axis=1)
    out_pad = pl.pallas_call(
        _epilogue_body,
        out_shape=jax.ShapeDtypeStruct((B, NPAD), jnp.float32),
    )(ssum_p, smax_p, h_pad, aux)
    return out_pad[:, :N]


# speedup vs baseline: 465.3205x; 1.5339x over previous
import functools

import jax
import jax.numpy as jnp
from jax import lax
from jax.experimental import pallas as pl
from jax.experimental.pallas import tpu as pltpu
from jax.experimental.pallas import tpu_sc as plsc

N = 10000
E = 320000
D = 128
R = 64
B = 2

NPAD = 10240
NTILE = 16
EPT = E // NTILE
ECHUNK = 4000
NCHUNK = EPT // ECHUNK
CVECS = ECHUNK // 16
UNROLL = 5
NODES_PER_TILE = NPAD // NTILE


def _prologue_body(ri_ref, rel_ref, w_ref, aux_ref):
    ri = ri_ref[...]
    rel = rel_ref[...]
    w = w_ref[...]
    iota_r = lax.broadcasted_iota(jnp.int32, (8, R), 1)
    oh = (ri[:, :R] == iota_r).astype(jnp.float32)
    query = jnp.dot(oh, rel, preferred_element_type=jnp.float32)
    wq = lax.dot_general(query, w, (((1,), (1,)), ((), ())),
                         preferred_element_type=jnp.float32)
    coef = lax.dot_general(query * wq, rel, (((1,), (1,)), ((), ())),
                           preferred_element_type=jnp.float32)
    q2 = jnp.sum(query * query, axis=1, keepdims=True)
    aux_ref[...] = jnp.concatenate(
        [coef, q2, jnp.zeros((8, D - R - 1), jnp.float32)], axis=1)


def _epilogue_body(ssum_ref, smax_ref, h_ref, aux_ref, out_ref):
    q2 = aux_ref[0:2, 64:65]
    score = ssum_ref[...] + h_ref[...] * q2
    neural = 1.0 / (1.0 + jnp.exp(-score))
    sym = jnp.maximum(smax_ref[...], 0.0)
    t = neural * sym
    out_ref[...] = jnp.log((t + 1e-10) / (1.0 - t + 1e-10))


def _sc_body(h_hbm, src_hbm, dst_hbm, et_hbm, aux_hbm, rb_hbm,
             osum_hbm, omax_hbm,
             hv, cv, rvv, srcva, dstva, etva, srcvb, dstvb, etvb,
             sumacc, maxacc, done_s,
             bsum, bmax, osum_v, omax_v, shsum, shmax, sema, semb):
    b = lax.axis_index("c")
    s = lax.axis_index("s")
    ebase = s * EPT

    bufsets = ((srcva, dstva, etva), (srcvb, dstvb, etvb))
    sems = (sema, semb)

    def start_chunk(ci):
        cbase = ebase + ci * ECHUNK
        bs = bufsets[ci % 2]
        sem = sems[ci % 2]
        return [pltpu.async_copy(src_hbm.at[pl.ds(cbase, ECHUNK)], bs[0], sem),
                pltpu.async_copy(dst_hbm.at[pl.ds(cbase, ECHUNK)], bs[1], sem),
                pltpu.async_copy(et_hbm.at[pl.ds(cbase, ECHUNK)], bs[2], sem)]

    handles = {0: start_chunk(0), 1: start_chunk(1)}

    pltpu.sync_copy(h_hbm.at[b], hv)
    pltpu.sync_copy(aux_hbm.at[b, pl.ds(0, R)], cv)
    pltpu.sync_copy(rb_hbm.at[b], rvv)

    zeros16 = jnp.zeros((16,), jnp.float32)
    neg16 = jnp.full((16,), -1e9, jnp.float32)

    def init_body(i, c):
        sumacc[pl.ds(i * 16, 16)] = zeros16
        maxacc[pl.ds(i * 16, 16)] = neg16
        return c

    lax.fori_loop(0, NPAD // 16, init_body, 0)

    rv = rvv[...]

    def edge_vec(sv, dv, tv, off):
        s16 = sv[pl.ds(off, 16)]
        d16 = dv[pl.ds(off, 16)]
        t16 = tv[pl.ds(off, 16)]
        hb = plsc.load_gather(hv, [s16])
        cf = plsc.load_gather(cv, [t16])
        plsc.addupdate_scatter(sumacc, [d16], hb * cf)
        m = t16 == rv
        cur = plsc.load_gather(maxacc, [d16])
        plsc.store_scatter(maxacc, [d16], jnp.maximum(cur, hb), mask=m)
        chk = plsc.load_gather(maxacc, [d16])
        return m & (chk < hb), d16, hb

    def fix_vec(pend, d16, hb):
        done_s[...] = jnp.where(pend, 0, 1)

        def retry(k, c2):
            p = done_s[...] == 0

            @pl.when(jnp.any(p))
            def _():
                cur2 = plsc.load_gather(maxacc, [d16])
                plsc.store_scatter(maxacc, [d16],
                                   jnp.maximum(cur2, hb), mask=p)
                chk2 = plsc.load_gather(maxacc, [d16])
                done_s[...] = jnp.where(p & (chk2 < hb), 0, 1)

            return c2

        lax.fori_loop(0, 15, retry, 0)

    def make_group_body(bufset):
        sv, dv, tv = bufset

        def group_body(i, c):
            base = i * (16 * UNROLL)
            res = [edge_vec(sv, dv, tv, base + u * 16) for u in range(UNROLL)]
            anyp = res[0][0]
            for r in res[1:]:
                anyp = anyp | r[0]

            @pl.when(jnp.any(anyp))
            def _():
                for r in res:
                    fix_vec(*r)

            return c

        return group_body

    for ci in range(NCHUNK):
        for hdl in handles.pop(ci):
            hdl.wait()
        lax.fori_loop(0, CVECS // UNROLL, make_group_body(bufsets[ci % 2]), 0)
        if ci + 2 < NCHUNK:
            handles[ci + 2] = start_chunk(ci + 2)

    pltpu.sync_copy(sumacc, shsum.at[s])
    pltpu.sync_copy(maxacc, shmax.at[s])
    plsc.subcore_barrier()

    nbase = s * NODES_PER_TILE
    pltpu.sync_copy(shsum.at[:, pl.ds(nbase, NODES_PER_TILE)], bsum)
    pltpu.sync_copy(shmax.at[:, pl.ds(nbase, NODES_PER_TILE)], bmax)

    def comb_body(k, c):
        off = k * 16
        a = bsum[0, pl.ds(off, 16)]
        mx = bmax[0, pl.ds(off, 16)]
        for j in range(1, NTILE):
            a = a + bsum[j, pl.ds(off, 16)]
            mx = jnp.maximum(mx, bmax[j, pl.ds(off, 16)])
        osum_v[pl.ds(off, 16)] = a
        omax_v[pl.ds(off, 16)] = mx
        return c

    lax.fori_loop(0, NODES_PER_TILE // 16, comb_body, 0)
    pltpu.sync_copy(osum_v, osum_hbm.at[b, pl.ds(nbase, NODES_PER_TILE)])
    pltpu.sync_copy(omax_v, omax_hbm.at[b, pl.ds(nbase, NODES_PER_TILE)])


def _build_sc_call():
    mesh = plsc.VectorSubcoreMesh(core_axis_name="c", subcore_axis_name="s")
    return pl.kernel(
        _sc_body,
        out_type=(jax.ShapeDtypeStruct((B, NPAD), jnp.float32),
                  jax.ShapeDtypeStruct((B, NPAD), jnp.float32)),
        mesh=mesh,
        compiler_params=pltpu.CompilerParams(needs_layout_passes=False),
        scratch_types=[
            pltpu.VMEM((N,), jnp.float32),
            pltpu.VMEM((R,), jnp.float32),
            pltpu.VMEM((16,), jnp.int32),
            pltpu.VMEM((ECHUNK,), jnp.int32),
            pltpu.VMEM((ECHUNK,), jnp.int32),
            pltpu.VMEM((ECHUNK,), jnp.int32),
            pltpu.VMEM((ECHUNK,), jnp.int32),
            pltpu.VMEM((ECHUNK,), jnp.int32),
            pltpu.VMEM((ECHUNK,), jnp.int32),
            pltpu.VMEM((NPAD,), jnp.float32),
            pltpu.VMEM((NPAD,), jnp.float32),
            pltpu.VMEM((16,), jnp.int32),
            pltpu.VMEM((NTILE, NODES_PER_TILE), jnp.float32),
            pltpu.VMEM((NTILE, NODES_PER_TILE), jnp.float32),
            pltpu.VMEM((NODES_PER_TILE,), jnp.float32),
            pltpu.VMEM((NODES_PER_TILE,), jnp.float32),
            pltpu.VMEM_SHARED((NTILE, NPAD), jnp.float32),
            pltpu.VMEM_SHARED((NTILE, NPAD), jnp.float32),
            pltpu.SemaphoreType.DMA,
            pltpu.SemaphoreType.DMA,
        ],
    )


def kernel(h_prob, edge_index, edge_type, r_index, rel_emb, W):
    src = edge_index[0]
    dst = edge_index[1]
    ri32 = r_index.astype(jnp.int32)
    rb16 = jnp.broadcast_to(ri32[:, None], (B, 16))
    ri8 = jnp.concatenate(
        [jnp.broadcast_to(ri32[:, None], (B, D)),
         jnp.zeros((8 - B, D), jnp.int32)], axis=0)

    aux = pl.pallas_call(
        _prologue_body,
        out_shape=jax.ShapeDtypeStruct((8, D), jnp.float32),
    )(ri8, rel_emb, W)

    ssum_p, smax_p = _build_sc_call()(h_prob, src, dst, edge_type, aux, rb16)

    h_pad = jnp.concatenate(
        [h_prob, jnp.zeros((B, NPAD - N), jnp.float32)], axis=1)
    out_pad = pl.pallas_call(
        _epilogue_body,
        out_shape=jax.ShapeDtypeStruct((B, NPAD), jnp.float32),
    )(ssum_p, smax_p, h_pad, aux)
    return out_pad[:, :N]
